# baseline (speedup 1.0000x reference)
import functools

import jax
import jax.numpy as jnp
from jax import lax
from jax.experimental import pallas as pl
from jax.experimental.pallas import tpu as pltpu
from jax.experimental.pallas import tpu_sc as plsc

N = 10000
D = 256
HID = 512
C = 40
CP = 48
E = 160000
EPAD = 163840
NC = 2
NS = 16
SUB = 128
NPT = N // NS
TRASH = N
ACCROWS = N + 8

_mesh = plsc.VectorSubcoreMesh(
    core_axis_name="c", subcore_axis_name="s", num_cores=NC, num_subcores=NS
)

_SROWS = 2 * EPAD // SUB
_DROWS = EPAD // SUB


@functools.partial(
    pl.kernel,
    out_type=jax.ShapeDtypeStruct((NC, N, 8), jnp.float32),
    mesh=_mesh,
    compiler_params=pltpu.CompilerParams(use_tc_tiling_on_sc=False),
    scratch_types=[
        pltpu.VMEM((5, 8 * SUB), jnp.int32),
        pltpu.VMEM((8 * SUB, 8), jnp.float32),
        pltpu.VMEM_SHARED((ACCROWS, 8), jnp.float32),
    ],
)
def _sc_deg(dst1k, zeros8, ones8, out, didx, ones_v, acc):
    c = lax.axis_index("c")
    s = lax.axis_index("s")
    w = c * NS + s
    pltpu.sync_copy(ones8, ones_v)
    pltpu.sync_copy(zeros8.at[pl.ds(s * NPT, NPT)], acc.at[pl.ds(s * NPT, NPT)])
    pltpu.sync_copy(dst1k.at[pl.ds(w * 5, 5)], didx)
    plsc.subcore_barrier()
    for k in range(5):
        pltpu.sync_copy(ones_v, acc.at[didx.at[k]], add=True)
    plsc.subcore_barrier()
    pltpu.sync_copy(acc.at[pl.ds(s * NPT, NPT)], out.at[c, pl.ds(s * NPT, NPT)])


def _edge_agg(table, src2d, dst2d, acc, sidx, didx, rows, semg, sems,
              nops, gi, src_row0, dst_row0):
    def group(g, carry):
        pltpu.sync_copy(src2d.at[pl.ds(src_row0 + g * gi, gi)], sidx)
        pltpu.sync_copy(dst2d.at[pl.ds(dst_row0 + g * gi, gi)], didx)
        nbuf = len(rows)
        gdesc = [None] * gi
        sdesc = [None] * gi
        gdesc[0] = pltpu.async_copy(table.at[sidx.at[0]], rows[0], semg[0])
        for k in range(gi):
            gdesc[k].wait()
            if k + 1 < gi:
                if k + 1 - nbuf >= 0:
                    sdesc[k + 1 - nbuf].wait()
                b = (k + 1) % nbuf
                gdesc[k + 1] = pltpu.async_copy(
                    table.at[sidx.at[k + 1]], rows[b], semg[b])
            sdesc[k] = pltpu.async_copy(rows[k % nbuf], acc.at[didx.at[k]],
                                        sems[k % nbuf], add=True)
        for t in range(max(0, gi - nbuf), gi):
            sdesc[t].wait()
        return carry

    lax.fori_loop(0, nops // gi, group, 0)


_NT1 = EPAD // NS // SUB
_GSZ1 = 16
_NT2 = EPAD // (NC * NS) // SUB


@functools.partial(
    pl.kernel,
    out_type=jax.ShapeDtypeStruct((4, N, 64), jnp.float32),
    mesh=_mesh,
    compiler_params=pltpu.CompilerParams(use_tc_tiling_on_sc=False),
    scratch_types=[
        pltpu.VMEM((8, 160), jnp.int32),
        pltpu.VMEM((8, 160), jnp.int32),
        [pltpu.VMEM((160, 64), jnp.float32)] * 4,
        [pltpu.SemaphoreType.DMA] * 4,
        [pltpu.SemaphoreType.DMA] * 4,
        pltpu.VMEM_SHARED((N, 64), jnp.float32),
        pltpu.VMEM_SHARED((ACCROWS, 64), jnp.float32),
    ],
)
def _sc_agg1(t1q, src2d, dst2d, out, sidx, didx, rows, semg, sems, tq, acc):
    c = lax.axis_index("c")
    s = lax.axis_index("s")
    for p in range(2):
        q = c * 2 + p
        pltpu.sync_copy(t1q.at[q, pl.ds(s * NPT, NPT)], tq.at[pl.ds(s * NPT, NPT)])
        pltpu.sync_copy(t1q.at[q, pl.ds(s * NPT, NPT)], acc.at[pl.ds(s * NPT, NPT)])
        plsc.subcore_barrier()
        _edge_agg(tq, src2d, dst2d, acc, sidx, didx, rows, semg, sems,
                  64, 8, s * 64, s * 64)
        plsc.subcore_barrier()
        pltpu.sync_copy(acc.at[pl.ds(s * NPT, NPT)], out.at[q, pl.ds(s * NPT, NPT)])
        if p == 0:
            plsc.subcore_barrier()


@functools.partial(
    pl.kernel,
    out_type=jax.ShapeDtypeStruct((NC, N, CP), jnp.float32),
    mesh=_mesh,
    compiler_params=pltpu.CompilerParams(use_tc_tiling_on_sc=False),
    scratch_types=[
        pltpu.VMEM((10, 4 * SUB), jnp.int32),
        pltpu.VMEM((10, 4 * SUB), jnp.int32),
        [pltpu.VMEM((4 * SUB, CP), jnp.float32)] * 2,
        [pltpu.SemaphoreType.DMA] * 2,
        [pltpu.SemaphoreType.DMA] * 2,
        pltpu.VMEM_SHARED((N, CP), jnp.float32),
        pltpu.VMEM_SHARED((ACCROWS, CP), jnp.float32),
    ],
)
def _sc_agg2(z2p, zeros48, src2d, dst2d, out, sidx, didx, rows, semg, sems, tq, acc):
    c = lax.axis_index("c")
    s = lax.axis_index("s")
    pltpu.sync_copy(z2p.at[pl.ds(s * NPT, NPT)], tq.at[pl.ds(s * NPT, NPT)])

    @pl.when(c == 0)
    def _():
        pltpu.sync_copy(z2p.at[pl.ds(s * NPT, NPT)], acc.at[pl.ds(s * NPT, NPT)])

    @pl.when(c != 0)
    def _():
        pltpu.sync_copy(zeros48.at[pl.ds(s * NPT, NPT)], acc.at[pl.ds(s * NPT, NPT)])

    plsc.subcore_barrier()
    _edge_agg(tq, src2d, dst2d, acc, sidx, didx, rows, semg, sems,
              10, 10, (c * NS + s) * 10, (c * NS + s) * 10)
    plsc.subcore_barrier()
    pltpu.sync_copy(acc.at[pl.ds(s * NPT, NPT)], out.at[c, pl.ds(s * NPT, NPT)])


def _dinv_from(degp_ref):
    deg = 1.0 + degp_ref[0, :, 0:1] + degp_ref[1, :, 0:1]
    return lax.rsqrt(deg)


def _tc_scale_body(x_ref, degp_ref, out_ref):
    p = pl.program_id(1)
    half = jnp.where(p == 0, x_ref[:, :64], x_ref[:, 64:])
    out_ref[...] = (half * _dinv_from(degp_ref))[None]


def _tc_mid_body(agg_ref, degp_ref, w1_ref, b1_ref, w2_ref, out_ref):
    dinv = _dinv_from(degp_ref)
    ax = jnp.concatenate([agg_ref[0], agg_ref[1], agg_ref[2],
                          agg_ref[3]], axis=1) * dinv
    h = jnp.dot(ax, w1_ref[...], preferred_element_type=jnp.float32) + b1_ref[...]
    h = jnp.maximum(h, 0.0)
    out_ref[...] = jnp.dot(h, w2_ref[...], preferred_element_type=jnp.float32) * dinv


def _tc_out_body(agg_ref, degp_ref, b2_ref, out_ref):
    dinv = _dinv_from(degp_ref)
    ssum = (agg_ref[0] + agg_ref[1]) * dinv
    logits = ssum[:, :C] + b2_ref[...]
    m = jnp.max(logits, axis=1, keepdims=True)
    lse = jnp.log(jnp.sum(jnp.exp(logits - m), axis=1, keepdims=True))
    out_ref[...] = logits - m - lse


_RS = 2000


def kernel(x, edge_index, W1, b1, W2, b2):
    src = edge_index[0].astype(jnp.int32)
    dst = edge_index[1].astype(jnp.int32)
    npad = EPAD - E
    src_p = jnp.concatenate([src, jnp.zeros((npad,), jnp.int32)])
    dst_p = jnp.concatenate([dst, jnp.full((npad,), TRASH, jnp.int32)])
    src256 = src_p.reshape(EPAD // 160, 160)
    dst256 = dst_p.reshape(EPAD // 160, 160)
    src512 = src_p.reshape(EPAD // 512, 512)
    dst512 = dst_p.reshape(EPAD // 512, 512)
    dst1k = dst_p.reshape(EPAD // 1024, 1024)
    zeros8 = jnp.zeros((N, 8), jnp.float32)
    ones8 = jnp.ones((8 * SUB, 8), jnp.float32)
    zeros48 = jnp.zeros((N, CP), jnp.float32)
    W2p = jnp.concatenate([W2, jnp.zeros((HID, CP - C), jnp.float32)], axis=1)

    degp = _sc_deg(dst1k, zeros8, ones8)

    t1q = pl.pallas_call(
        _tc_scale_body,
        grid=(2, 2, N // _RS),
        in_specs=[
            pl.BlockSpec((_RS, 128), lambda c, p, i: (i, c)),
            pl.BlockSpec((NC, _RS, 8), lambda c, p, i: (0, i, 0)),
        ],
        out_specs=pl.BlockSpec((1, _RS, 64), lambda c, p, i: (c * 2 + p, i, 0)),
        out_shape=jax.ShapeDtypeStruct((4, N, 64), jnp.float32),
    )(x, degp)

    agg1 = _sc_agg1(t1q, src256, dst256)

    z2p = pl.pallas_call(
        _tc_mid_body,
        grid=(N // _RS,),
        in_specs=[
            pl.BlockSpec((4, _RS, 64), lambda i: (0, i, 0)),
            pl.BlockSpec((NC, _RS, 8), lambda i: (0, i, 0)),
            pl.BlockSpec((D, HID), lambda i: (0, 0)),
            pl.BlockSpec((1, HID), lambda i: (0, 0)),
            pl.BlockSpec((HID, CP), lambda i: (0, 0)),
        ],
        out_specs=pl.BlockSpec((_RS, CP), lambda i: (i, 0)),
        out_shape=jax.ShapeDtypeStruct((N, CP), jnp.float32),
    )(agg1, degp, W1, b1.reshape(1, HID), W2p)

    agg2 = _sc_agg2(z2p, zeros48, src512, dst512)

    out = pl.pallas_call(
        _tc_out_body,
        grid=(N // _RS,),
        in_specs=[
            pl.BlockSpec((NC, _RS, CP), lambda i: (0, i, 0)),
            pl.BlockSpec((NC, _RS, 8), lambda i: (0, i, 0)),
            pl.BlockSpec((1, C), lambda i: (0, 0)),
        ],
        out_specs=pl.BlockSpec((_RS, C), lambda i: (i, 0)),
        out_shape=jax.ShapeDtypeStruct((N, C), jnp.float32),
    )(agg2, degp, b2.reshape(1, C))

    return out


# speedup vs baseline: 1.0460x; 1.0460x over previous
import functools

import jax
import jax.numpy as jnp
from jax import lax
from jax.experimental import pallas as pl
from jax.experimental.pallas import tpu as pltpu
from jax.experimental.pallas import tpu_sc as plsc

N = 10000
D = 256
HID = 512
C = 40
CP = 48
E = 160000
EPAD = 163840
NC = 2
NS = 16
SUB = 128
NPT = N // NS
TRASH = N
ACCROWS = N + 8

_mesh = plsc.VectorSubcoreMesh(
    core_axis_name="c", subcore_axis_name="s", num_cores=NC, num_subcores=NS
)

_SROWS = 2 * EPAD // SUB
_DROWS = EPAD // SUB


@functools.partial(
    pl.kernel,
    out_type=jax.ShapeDtypeStruct((NC, N, 8), jnp.float32),
    mesh=_mesh,
    compiler_params=pltpu.CompilerParams(use_tc_tiling_on_sc=False),
    scratch_types=[
        pltpu.VMEM((5, 8 * SUB), jnp.int32),
        pltpu.VMEM((8 * SUB, 8), jnp.float32),
        pltpu.VMEM_SHARED((ACCROWS, 8), jnp.float32),
    ],
)
def _sc_deg(dst1k, zeros8, ones8, out, didx, ones_v, acc):
    c = lax.axis_index("c")
    s = lax.axis_index("s")
    w = c * NS + s
    pltpu.sync_copy(ones8, ones_v)
    pltpu.sync_copy(zeros8.at[pl.ds(s * NPT, NPT)], acc.at[pl.ds(s * NPT, NPT)])
    pltpu.sync_copy(dst1k.at[pl.ds(w * 5, 5)], didx)
    plsc.subcore_barrier()
    for k in range(5):
        pltpu.sync_copy(ones_v, acc.at[didx.at[k]], add=True)
    plsc.subcore_barrier()
    pltpu.sync_copy(acc.at[pl.ds(s * NPT, NPT)], out.at[c, pl.ds(s * NPT, NPT)])


def _edge_agg(table, src2d, dst2d, acc, sidx, didx, rows, semg, sems,
              nops, gi, src_row0, dst_row0):
    def group(g, carry):
        pltpu.sync_copy(src2d.at[pl.ds(src_row0 + g * gi, gi)], sidx)
        pltpu.sync_copy(dst2d.at[pl.ds(dst_row0 + g * gi, gi)], didx)
        nbuf = len(rows)
        gdesc = [None] * gi
        sdesc = [None] * gi
        gdesc[0] = pltpu.async_copy(table.at[sidx.at[0]], rows[0], semg[0])
        for k in range(gi):
            gdesc[k].wait()
            if k + 1 < gi:
                if k + 1 - nbuf >= 0:
                    sdesc[k + 1 - nbuf].wait()
                b = (k + 1) % nbuf
                gdesc[k + 1] = pltpu.async_copy(
                    table.at[sidx.at[k + 1]], rows[b], semg[b])
            sdesc[k] = pltpu.async_copy(rows[k % nbuf], acc.at[didx.at[k]],
                                        sems[k % nbuf], add=True)
        for t in range(max(0, gi - nbuf), gi):
            sdesc[t].wait()
        return carry

    lax.fori_loop(0, nops // gi, group, 0)


_NT1 = EPAD // NS // SUB
_GSZ1 = 16
_NT2 = EPAD // (NC * NS) // SUB


@functools.partial(
    pl.kernel,
    out_type=jax.ShapeDtypeStruct((4, N, 64), jnp.float32),
    mesh=_mesh,
    compiler_params=pltpu.CompilerParams(use_tc_tiling_on_sc=False),
    scratch_types=[
        pltpu.VMEM((8, 320), jnp.int32),
        pltpu.VMEM((8, 320), jnp.int32),
        [pltpu.VMEM((320, 64), jnp.float32)] * 2,
        [pltpu.SemaphoreType.DMA] * 2,
        [pltpu.SemaphoreType.DMA] * 2,
        pltpu.VMEM_SHARED((N, 64), jnp.float32),
        pltpu.VMEM_SHARED((ACCROWS, 64), jnp.float32),
    ],
)
def _sc_agg1(t1q, src2d, dst2d, out, sidx, didx, rows, semg, sems, tq, acc):
    c = lax.axis_index("c")
    s = lax.axis_index("s")
    for p in range(2):
        q = c * 2 + p
        pltpu.sync_copy(t1q.at[q, pl.ds(s * NPT, NPT)], tq.at[pl.ds(s * NPT, NPT)])
        pltpu.sync_copy(t1q.at[q, pl.ds(s * NPT, NPT)], acc.at[pl.ds(s * NPT, NPT)])
        plsc.subcore_barrier()
        _edge_agg(tq, src2d, dst2d, acc, sidx, didx, rows, semg, sems,
                  32, 8, s * 32, s * 32)
        plsc.subcore_barrier()
        pltpu.sync_copy(acc.at[pl.ds(s * NPT, NPT)], out.at[q, pl.ds(s * NPT, NPT)])
        if p == 0:
            plsc.subcore_barrier()


@functools.partial(
    pl.kernel,
    out_type=jax.ShapeDtypeStruct((NC, N, CP), jnp.float32),
    mesh=_mesh,
    compiler_params=pltpu.CompilerParams(use_tc_tiling_on_sc=False),
    scratch_types=[
        pltpu.VMEM((10, 4 * SUB), jnp.int32),
        pltpu.VMEM((10, 4 * SUB), jnp.int32),
        [pltpu.VMEM((4 * SUB, CP), jnp.float32)] * 2,
        [pltpu.SemaphoreType.DMA] * 2,
        [pltpu.SemaphoreType.DMA] * 2,
        pltpu.VMEM_SHARED((N, CP), jnp.float32),
        pltpu.VMEM_SHARED((ACCROWS, CP), jnp.float32),
    ],
)
def _sc_agg2(z2p, zeros48, src2d, dst2d, out, sidx, didx, rows, semg, sems, tq, acc):
    c = lax.axis_index("c")
    s = lax.axis_index("s")
    pltpu.sync_copy(z2p.at[pl.ds(s * NPT, NPT)], tq.at[pl.ds(s * NPT, NPT)])

    @pl.when(c == 0)
    def _():
        pltpu.sync_copy(z2p.at[pl.ds(s * NPT, NPT)], acc.at[pl.ds(s * NPT, NPT)])

    @pl.when(c != 0)
    def _():
        pltpu.sync_copy(zeros48.at[pl.ds(s * NPT, NPT)], acc.at[pl.ds(s * NPT, NPT)])

    plsc.subcore_barrier()
    _edge_agg(tq, src2d, dst2d, acc, sidx, didx, rows, semg, sems,
              10, 10, (c * NS + s) * 10, (c * NS + s) * 10)
    plsc.subcore_barrier()
    pltpu.sync_copy(acc.at[pl.ds(s * NPT, NPT)], out.at[c, pl.ds(s * NPT, NPT)])


def _dinv_from(degp_ref):
    deg = 1.0 + degp_ref[0, :, 0:1] + degp_ref[1, :, 0:1]
    return lax.rsqrt(deg)


def _tc_scale_body(x_ref, degp_ref, out_ref):
    p = pl.program_id(1)
    half = jnp.where(p == 0, x_ref[:, :64], x_ref[:, 64:])
    out_ref[...] = (half * _dinv_from(degp_ref))[None]


def _tc_mid_body(agg_ref, degp_ref, w1_ref, b1_ref, w2_ref, out_ref):
    dinv = _dinv_from(degp_ref)
    ax = jnp.concatenate([agg_ref[0], agg_ref[1], agg_ref[2],
                          agg_ref[3]], axis=1) * dinv
    h = jnp.dot(ax, w1_ref[...], preferred_element_type=jnp.float32) + b1_ref[...]
    h = jnp.maximum(h, 0.0)
    out_ref[...] = jnp.dot(h, w2_ref[...], preferred_element_type=jnp.float32) * dinv


def _tc_out_body(agg_ref, degp_ref, b2_ref, out_ref):
    dinv = _dinv_from(degp_ref)
    ssum = (agg_ref[0] + agg_ref[1]) * dinv
    logits = ssum[:, :C] + b2_ref[...]
    m = jnp.max(logits, axis=1, keepdims=True)
    lse = jnp.log(jnp.sum(jnp.exp(logits - m), axis=1, keepdims=True))
    out_ref[...] = logits - m - lse


_RS = 2000


def kernel(x, edge_index, W1, b1, W2, b2):
    src = edge_index[0].astype(jnp.int32)
    dst = edge_index[1].astype(jnp.int32)
    npad = EPAD - E
    src_p = jnp.concatenate([src, jnp.zeros((npad,), jnp.int32)])
    dst_p = jnp.concatenate([dst, jnp.full((npad,), TRASH, jnp.int32)])
    src256 = src_p.reshape(EPAD // 320, 320)
    dst256 = dst_p.reshape(EPAD // 320, 320)
    src512 = src_p.reshape(EPAD // 512, 512)
    dst512 = dst_p.reshape(EPAD // 512, 512)
    dst1k = dst_p.reshape(EPAD // 1024, 1024)
    zeros8 = jnp.zeros((N, 8), jnp.float32)
    ones8 = jnp.ones((8 * SUB, 8), jnp.float32)
    zeros48 = jnp.zeros((N, CP), jnp.float32)
    W2p = jnp.concatenate([W2, jnp.zeros((HID, CP - C), jnp.float32)], axis=1)

    degp = _sc_deg(dst1k, zeros8, ones8)

    t1q = pl.pallas_call(
        _tc_scale_body,
        grid=(2, 2, N // _RS),
        in_specs=[
            pl.BlockSpec((_RS, 128), lambda c, p, i: (i, c)),
            pl.BlockSpec((NC, _RS, 8), lambda c, p, i: (0, i, 0)),
        ],
        out_specs=pl.BlockSpec((1, _RS, 64), lambda c, p, i: (c * 2 + p, i, 0)),
        out_shape=jax.ShapeDtypeStruct((4, N, 64), jnp.float32),
    )(x, degp)

    agg1 = _sc_agg1(t1q, src256, dst256)

    z2p = pl.pallas_call(
        _tc_mid_body,
        grid=(N // _RS,),
        in_specs=[
            pl.BlockSpec((4, _RS, 64), lambda i: (0, i, 0)),
            pl.BlockSpec((NC, _RS, 8), lambda i: (0, i, 0)),
            pl.BlockSpec((D, HID), lambda i: (0, 0)),
            pl.BlockSpec((1, HID), lambda i: (0, 0)),
            pl.BlockSpec((HID, CP), lambda i: (0, 0)),
        ],
        out_specs=pl.BlockSpec((_RS, CP), lambda i: (i, 0)),
        out_shape=jax.ShapeDtypeStruct((N, CP), jnp.float32),
    )(agg1, degp, W1, b1.reshape(1, HID), W2p)

    agg2 = _sc_agg2(z2p, zeros48, src512, dst512)

    out = pl.pallas_call(
        _tc_out_body,
        grid=(N // _RS,),
        in_specs=[
            pl.BlockSpec((NC, _RS, CP), lambda i: (0, i, 0)),
            pl.BlockSpec((NC, _RS, 8), lambda i: (0, i, 0)),
            pl.BlockSpec((1, C), lambda i: (0, 0)),
        ],
        out_specs=pl.BlockSpec((_RS, C), lambda i: (i, 0)),
        out_shape=jax.ShapeDtypeStruct((N, C), jnp.float32),
    )(agg2, degp, b2.reshape(1, C))

    return out


# speedup vs baseline: 1.0468x; 1.0007x over previous
import functools

import jax
import jax.numpy as jnp
from jax import lax
from jax.experimental import pallas as pl
from jax.experimental.pallas import tpu as pltpu
from jax.experimental.pallas import tpu_sc as plsc

N = 10000
D = 256
HID = 512
C = 40
CP = 48
E = 160000
EPAD = 163840
NC = 2
NS = 16
SUB = 128
NPT = N // NS
TRASH = N
ACCROWS = N + 8

_mesh = plsc.VectorSubcoreMesh(
    core_axis_name="c", subcore_axis_name="s", num_cores=NC, num_subcores=NS
)


@functools.partial(
    pl.kernel,
    out_type=jax.ShapeDtypeStruct((NC, N, 8), jnp.float32),
    mesh=_mesh,
    compiler_params=pltpu.CompilerParams(use_tc_tiling_on_sc=False),
    scratch_types=[
        pltpu.VMEM((5, 8 * SUB), jnp.int32),
        pltpu.VMEM((8 * SUB, 8), jnp.float32),
        pltpu.VMEM_SHARED((ACCROWS, 8), jnp.float32),
    ],
)
def _sc_deg(dst1k, zeros8, ones8, out, didx, ones_v, acc):
    c = lax.axis_index("c")
    s = lax.axis_index("s")
    w = c * NS + s
    pltpu.sync_copy(ones8, ones_v)
    pltpu.sync_copy(zeros8.at[pl.ds(s * NPT, NPT)], acc.at[pl.ds(s * NPT, NPT)])
    pltpu.sync_copy(dst1k.at[pl.ds(w * 5, 5)], didx)
    plsc.subcore_barrier()
    for k in range(5):
        pltpu.sync_copy(ones_v, acc.at[didx.at[k]], add=True)
    plsc.subcore_barrier()
    pltpu.sync_copy(acc.at[pl.ds(s * NPT, NPT)], out.at[c, pl.ds(s * NPT, NPT)])


def _edge_agg(table, src2d, dst2d, acc, sidx, didx, rows, semg, sems,
              nops, gi, src_row0, dst_row0):
    def group(g, carry):
        pltpu.sync_copy(src2d.at[pl.ds(src_row0 + g * gi, gi)], sidx)
        pltpu.sync_copy(dst2d.at[pl.ds(dst_row0 + g * gi, gi)], didx)
        nbuf = len(rows)
        gdesc = [None] * gi
        sdesc = [None] * gi
        gdesc[0] = pltpu.async_copy(table.at[sidx.at[0]], rows[0], semg[0])
        for k in range(gi):
            gdesc[k].wait()
            if k + 1 < gi:
                if k + 1 - nbuf >= 0:
                    sdesc[k + 1 - nbuf].wait()
                b = (k + 1) % nbuf
                gdesc[k + 1] = pltpu.async_copy(
                    table.at[sidx.at[k + 1]], rows[b], semg[b])
            sdesc[k] = pltpu.async_copy(rows[k % nbuf], acc.at[didx.at[k]],
                                        sems[k % nbuf], add=True)
        for t in range(max(0, gi - nbuf), gi):
            sdesc[t].wait()
        return carry

    lax.fori_loop(0, nops // gi, group, 0)


@functools.partial(
    pl.kernel,
    out_type=jax.ShapeDtypeStruct((4, N, 64), jnp.float32),
    mesh=_mesh,
    compiler_params=pltpu.CompilerParams(use_tc_tiling_on_sc=False),
    scratch_types=[
        pltpu.VMEM((8, 320), jnp.int32),
        pltpu.VMEM((8, 320), jnp.int32),
        [pltpu.VMEM((320, 64), jnp.float32)] * 2,
        [pltpu.SemaphoreType.DMA] * 2,
        [pltpu.SemaphoreType.DMA] * 2,
        pltpu.VMEM_SHARED((N, 64), jnp.float32),
        pltpu.VMEM_SHARED((ACCROWS, 64), jnp.float32),
    ],
)
def _sc_agg1(t1q, src2d, dst2d, out, sidx, didx, rows, semg, sems, tq, acc):
    c = lax.axis_index("c")
    s = lax.axis_index("s")
    for p in range(2):
        q = c * 2 + p
        pltpu.sync_copy(t1q.at[q, pl.ds(s * NPT, NPT)], tq.at[pl.ds(s * NPT, NPT)])
        pltpu.sync_copy(t1q.at[q, pl.ds(s * NPT, NPT)], acc.at[pl.ds(s * NPT, NPT)])
        plsc.subcore_barrier()
        _edge_agg(tq, src2d, dst2d, acc, sidx, didx, rows, semg, sems,
                  32, 8, s * 32, s * 32)
        plsc.subcore_barrier()
        pltpu.sync_copy(acc.at[pl.ds(s * NPT, NPT)], out.at[q, pl.ds(s * NPT, NPT)])
        if p == 0:
            plsc.subcore_barrier()


@functools.partial(
    pl.kernel,
    out_type=jax.ShapeDtypeStruct((NC, N, CP), jnp.float32),
    mesh=_mesh,
    compiler_params=pltpu.CompilerParams(use_tc_tiling_on_sc=False),
    scratch_types=[
        pltpu.VMEM((10, 4 * SUB), jnp.int32),
        pltpu.VMEM((10, 4 * SUB), jnp.int32),
        [pltpu.VMEM((4 * SUB, CP), jnp.float32)] * 2,
        [pltpu.SemaphoreType.DMA] * 2,
        [pltpu.SemaphoreType.DMA] * 2,
        pltpu.VMEM_SHARED((N, CP), jnp.float32),
        pltpu.VMEM_SHARED((ACCROWS, CP), jnp.float32),
    ],
)
def _sc_agg2(z2p, zeros48, src2d, dst2d, out, sidx, didx, rows, semg, sems, tq, acc):
    c = lax.axis_index("c")
    s = lax.axis_index("s")
    pltpu.sync_copy(z2p.at[pl.ds(s * NPT, NPT)], tq.at[pl.ds(s * NPT, NPT)])

    @pl.when(c == 0)
    def _():
        pltpu.sync_copy(z2p.at[pl.ds(s * NPT, NPT)], acc.at[pl.ds(s * NPT, NPT)])

    @pl.when(c != 0)
    def _():
        pltpu.sync_copy(zeros48.at[pl.ds(s * NPT, NPT)], acc.at[pl.ds(s * NPT, NPT)])

    plsc.subcore_barrier()
    _edge_agg(tq, src2d, dst2d, acc, sidx, didx, rows, semg, sems,
              10, 10, (c * NS + s) * 10, (c * NS + s) * 10)
    plsc.subcore_barrier()
    pltpu.sync_copy(acc.at[pl.ds(s * NPT, NPT)], out.at[c, pl.ds(s * NPT, NPT)])


def _dinv_from(degp_ref):
    deg = 1.0 + degp_ref[0, :, 0:1] + degp_ref[1, :, 0:1]
    return lax.rsqrt(deg)


def _tc_scale_body(x_ref, degp_ref, out_ref):
    p = pl.program_id(1)
    half = jnp.where(p == 0, x_ref[:, :64], x_ref[:, 64:])
    out_ref[...] = (half * _dinv_from(degp_ref))[None]


def _tc_mid_body(agg_ref, degp_ref, w1_ref, b1_ref, w2_ref, out_ref):
    dinv = _dinv_from(degp_ref)
    ax = jnp.concatenate([agg_ref[0], agg_ref[1], agg_ref[2],
                          agg_ref[3]], axis=1) * dinv
    h = jnp.dot(ax, w1_ref[...], preferred_element_type=jnp.float32) + b1_ref[...]
    h = jnp.maximum(h, 0.0)
    out_ref[...] = jnp.dot(h, w2_ref[...], preferred_element_type=jnp.float32) * dinv


def _tc_out_body(agg_ref, degp_ref, b2_ref, out_ref):
    dinv = _dinv_from(degp_ref)
    ssum = (agg_ref[0] + agg_ref[1]) * dinv
    logits = ssum[:, :C] + b2_ref[...]
    m = jnp.max(logits, axis=1, keepdims=True)
    lse = jnp.log(jnp.sum(jnp.exp(logits - m), axis=1, keepdims=True))
    out_ref[...] = logits - m - lse


_RS = 2000


def kernel(x, edge_index, W1, b1, W2, b2):
    src = edge_index[0].astype(jnp.int32)
    dst = edge_index[1].astype(jnp.int32)
    npad = EPAD - E
    src_p = jnp.concatenate([src, jnp.zeros((npad,), jnp.int32)])
    dst_p = jnp.concatenate([dst, jnp.full((npad,), TRASH, jnp.int32)])
    src256 = src_p.reshape(EPAD // 320, 320)
    dst256 = dst_p.reshape(EPAD // 320, 320)
    src512 = src_p.reshape(EPAD // 512, 512)
    dst512 = dst_p.reshape(EPAD // 512, 512)
    dst1k = dst_p.reshape(EPAD // 1024, 1024)
    zeros8 = jnp.zeros((N, 8), jnp.float32)
    ones8 = jnp.ones((8 * SUB, 8), jnp.float32)
    zeros48 = jnp.zeros((N, CP), jnp.float32)
    W2p = jnp.concatenate([W2, jnp.zeros((HID, CP - C), jnp.float32)], axis=1)

    degp = _sc_deg(dst1k, zeros8, ones8)

    t1q = pl.pallas_call(
        _tc_scale_body,
        grid=(2, 2, N // _RS),
        in_specs=[
            pl.BlockSpec((_RS, 128), lambda c, p, i: (i, c)),
            pl.BlockSpec((NC, _RS, 8), lambda c, p, i: (0, i, 0)),
        ],
        out_specs=pl.BlockSpec((1, _RS, 64), lambda c, p, i: (c * 2 + p, i, 0)),
        out_shape=jax.ShapeDtypeStruct((4, N, 64), jnp.float32),
    )(x, degp)

    agg1 = _sc_agg1(t1q, src256, dst256)

    z2p = pl.pallas_call(
        _tc_mid_body,
        grid=(N // _RS,),
        in_specs=[
            pl.BlockSpec((4, _RS, 64), lambda i: (0, i, 0)),
            pl.BlockSpec((NC, _RS, 8), lambda i: (0, i, 0)),
            pl.BlockSpec((D, HID), lambda i: (0, 0)),
            pl.BlockSpec((1, HID), lambda i: (0, 0)),
            pl.BlockSpec((HID, CP), lambda i: (0, 0)),
        ],
        out_specs=pl.BlockSpec((_RS, CP), lambda i: (i, 0)),
        out_shape=jax.ShapeDtypeStruct((N, CP), jnp.float32),
    )(agg1, degp, W1, b1.reshape(1, HID), W2p)

    agg2 = _sc_agg2(z2p, zeros48, src512, dst512)

    out = pl.pallas_call(
        _tc_out_body,
        grid=(N // _RS,),
        in_specs=[
            pl.BlockSpec((NC, _RS, CP), lambda i: (0, i, 0)),
            pl.BlockSpec((NC, _RS, 8), lambda i: (0, i, 0)),
            pl.BlockSpec((1, C), lambda i: (0, 0)),
        ],
        out_specs=pl.BlockSpec((_RS, C), lambda i: (i, 0)),
        out_shape=jax.ShapeDtypeStruct((N, C), jnp.float32),
    )(agg2, degp, b2.reshape(1, C))

    return out
